# Initial kernel scaffold; baseline (speedup 1.0000x reference)
import jax
import jax.numpy as jnp
from jax.experimental import pallas as pl


def kernel(labels, mean, log_var, domain_weights):
    raise NotImplementedError("write your pallas kernel here")


# speedup vs baseline: 262.3040x; 262.3040x over previous
import functools

import numpy as np
import jax
import jax.numpy as jnp
from jax import lax
from jax.experimental import pallas as pl
from jax.experimental.pallas import tpu as pltpu
from jax.experimental.pallas import tpu_sc as plsc

_B = 16384
_D = 26
_K = 1000
_RW = 32
_GB = 8
_GROUP = _GB * _D
_OUTW = _GB * _D * _D
_NW = 32
_BPW = _B // _NW
_NG = _BPW // _GB

_n = np.arange(_GROUP, dtype=np.int32)
_JPAT = np.asarray((_n % _D) * _K, dtype=np.int32)
_m = np.arange(_OUTW, dtype=np.int32)
_TPR = np.asarray((_m // (_D * _D)) * _D + (_m % _D), dtype=np.int32)
_TPC = np.asarray((_m % (_D * _D)) // _D, dtype=np.int32)


def _sc_gather(table, labels_flat, jpat, tpr, tpc):
    mesh = plsc.VectorSubcoreMesh(core_axis_name="c", subcore_axis_name="s")

    @functools.partial(
        pl.kernel,
        out_type=[
            jax.ShapeDtypeStruct((_B * _D * _D,), jnp.float32),
            jax.ShapeDtypeStruct((_B * _D,), jnp.float32),
            jax.ShapeDtypeStruct((_B * _D,), jnp.float32),
        ],
        mesh=mesh,
        compiler_params=pltpu.CompilerParams(
            needs_layout_passes=False, use_tc_tiling_on_sc=False),
        scratch_types=[
            pltpu.VMEM((_GROUP,), jnp.int32),
            pltpu.VMEM((_GROUP,), jnp.int32),
            pltpu.VMEM((_GROUP, _RW), jnp.float32),
            pltpu.VMEM((_OUTW,), jnp.float32),
            pltpu.VMEM((_GROUP,), jnp.float32),
            pltpu.VMEM((_GROUP,), jnp.float32),
            pltpu.VMEM((_GROUP,), jnp.int32),
            pltpu.VMEM((_OUTW,), jnp.int32),
            pltpu.VMEM((_OUTW,), jnp.int32),
            pltpu.SemaphoreType.DMA,
        ],
    )
    def k(table_hbm, lab_hbm, jpat_hbm, tpr_hbm, tpc_hbm,
          dw_hbm, mean_hbm, lv_hbm,
          lbuf, idxbuf, rbuf, obuf, mbuf, vbuf, jpat_v, tpr_v, tpc_v, sem):
        wid = lax.axis_index("s") * 2 + lax.axis_index("c")
        b0 = wid * _BPW
        pltpu.sync_copy(jpat_hbm, jpat_v)
        pltpu.sync_copy(tpr_hbm, tpr_v)
        pltpu.sync_copy(tpc_hbm, tpc_v)
        lanes = lax.iota(jnp.int32, 16)
        c_mean = jnp.full((16,), _D, jnp.int32)
        c_lv = jnp.full((16,), _D + 1, jnp.int32)

        def group(g, carry):
            base_b = b0 + g * _GB
            pltpu.sync_copy(lab_hbm.at[pl.ds(base_b * _D, _GROUP)], lbuf)

            def mkidx(v, c):
                s = pl.ds(v * 16, 16)
                idxbuf[s] = lbuf[s] + jpat_v[s]
                return c
            lax.fori_loop(0, _GROUP // 16, mkidx, 0)

            half = _GROUP // 2
            cp1 = pltpu.async_copy(
                table_hbm.at[idxbuf.at[pl.ds(0, half)]],
                rbuf.at[pl.ds(0, half)], sem)
            cp2 = pltpu.async_copy(
                table_hbm.at[idxbuf.at[pl.ds(half, half)]],
                rbuf.at[pl.ds(half, half)], sem)
            cp1.wait()
            cp2.wait()

            def tr(v, c):
                s = pl.ds(v * 16, 16)
                obuf[s] = plsc.load_gather(rbuf, [tpr_v[s], tpc_v[s]])
                return c
            lax.fori_loop(0, _OUTW // 16, tr, 0)

            def mv(v, c):
                s = pl.ds(v * 16, 16)
                rows = lanes + v * 16
                mbuf[s] = plsc.load_gather(rbuf, [rows, c_mean])
                vbuf[s] = plsc.load_gather(rbuf, [rows, c_lv])
                return c
            lax.fori_loop(0, _GROUP // 16, mv, 0)

            pltpu.sync_copy(obuf, dw_hbm.at[pl.ds(base_b * _D * _D, _OUTW)])
            pltpu.sync_copy(mbuf, mean_hbm.at[pl.ds(base_b * _D, _GROUP)])
            pltpu.sync_copy(vbuf, lv_hbm.at[pl.ds(base_b * _D, _GROUP)])
            return carry
        lax.fori_loop(0, _NG, group, 0)

    return k(table, labels_flat, jpat, tpr, tpc)


def kernel(labels, mean, log_var, domain_weights):
    labels = labels.astype(jnp.int32)
    tdw = jnp.transpose(domain_weights, (1, 2, 0))
    table = jnp.concatenate(
        [tdw, mean[:, :, None], log_var[:, :, None],
         jnp.zeros((_D, _K, _RW - _D - 2), jnp.float32)], axis=2)
    table = table.reshape(_D * _K, _RW)
    dwf, mf, vf = _sc_gather(
        table, labels.reshape(-1),
        jnp.asarray(_JPAT), jnp.asarray(_TPR), jnp.asarray(_TPC))
    return (mf.reshape(_B, _D), vf.reshape(_B, _D),
            dwf.reshape(_B, _D, _D))


# speedup vs baseline: 456.7349x; 1.7412x over previous
import functools

import numpy as np
import jax
import jax.numpy as jnp
from jax import lax
from jax.experimental import pallas as pl
from jax.experimental.pallas import tpu as pltpu
from jax.experimental.pallas import tpu_sc as plsc

_B = 16384
_D = 26
_K = 1000
_RW = 32
_GB = 16
_GROUP = _GB * _D
_OUTW = _GB * _D * _D
_NW = 32
_BPW = _B // _NW
_NG = _BPW // _GB
_GCH = 104

_n = np.arange(_GROUP, dtype=np.int32)
_JPAT = np.asarray((_n % _D) * _K, dtype=np.int32)


def _sc_gather(table, labels_flat, jpat):
    mesh = plsc.VectorSubcoreMesh(core_axis_name="c", subcore_axis_name="s")

    @functools.partial(
        pl.kernel,
        out_type=[
            jax.ShapeDtypeStruct((_B * _D * _D,), jnp.float32),
            jax.ShapeDtypeStruct((_B * _D,), jnp.float32),
            jax.ShapeDtypeStruct((_B * _D,), jnp.float32),
        ],
        mesh=mesh,
        compiler_params=pltpu.CompilerParams(
            needs_layout_passes=False, use_tc_tiling_on_sc=False),
        scratch_types=(
            [pltpu.VMEM((_GROUP,), jnp.int32)] * 2
            + [pltpu.VMEM((_GROUP,), jnp.int32)] * 2
            + [pltpu.VMEM((_GROUP, _RW), jnp.float32)] * 2
            + [pltpu.VMEM((_OUTW,), jnp.float32)] * 2
            + [pltpu.VMEM((_GROUP,), jnp.float32)] * 2
            + [pltpu.VMEM((_GROUP,), jnp.float32)] * 2
            + [pltpu.VMEM((_GROUP,), jnp.int32)]
            + [pltpu.SemaphoreType.DMA] * 6
        ),
    )
    def k(table_hbm, lab_hbm, jpat_hbm,
          dw_hbm, mean_hbm, lv_hbm,
          lbuf0, lbuf1, idx0, idx1, rbuf0, rbuf1, obuf0, obuf1,
          mbuf0, mbuf1, vbuf0, vbuf1, jpat_v,
          sl0, sl1, sg0, sg1, so0, so1):
        lbuf = (lbuf0, lbuf1)
        idxb = (idx0, idx1)
        rbuf = (rbuf0, rbuf1)
        obuf = (obuf0, obuf1)
        mbuf = (mbuf0, mbuf1)
        vbuf = (vbuf0, vbuf1)
        slab = (sl0, sl1)
        sgat = (sg0, sg1)
        sout = (so0, so1)

        wid = lax.axis_index("s") * 2 + lax.axis_index("c")
        b0 = wid * _BPW
        pltpu.sync_copy(jpat_hbm, jpat_v)
        lanes = lax.iota(jnp.int32, 16)
        cvec = lanes * _D
        c2vec = cvec + 16 * _D
        m10 = lanes < (_D - 16)
        col_mean = jnp.full((16,), _D, jnp.int32)
        col_lv = jnp.full((16,), _D + 1, jnp.int32)

        def lab_slice(g):
            return lab_hbm.at[pl.ds((b0 + g * _GB) * _D, _GROUP)]

        def fire_labels(g, p):
            return pltpu.async_copy(lab_slice(g), lbuf[p], slab[p])

        def wait_labels(g, p):
            pltpu.make_async_copy(lab_slice(g), lbuf[p], slab[p]).wait()

        def compute_idx(p):
            def body(v, c):
                s = pl.ds(v * 16, 16)
                idxb[p][s] = lbuf[p][s] + jpat_v[s]
                return c
            lax.fori_loop(0, _GROUP // 16, body, 0)

        def fire_gathers(p):
            for c in range(_GROUP // _GCH):
                s = pl.ds(c * _GCH, _GCH)
                pltpu.async_copy(table_hbm.at[idxb[p].at[s]],
                                 rbuf[p].at[s], sgat[p])

        def wait_gathers(p):
            for c in range(_GROUP // _GCH):
                s = pl.ds(c * _GCH, _GCH)
                pltpu.make_async_copy(table_hbm.at[idxb[p].at[s]],
                                      rbuf[p].at[s], sgat[p]).wait()

        def transpose(p):
            def bl_body(bl, c):
                obase = bl * (_D * _D)
                rbase = bl * _D
                for j in range(_D):
                    v1 = rbuf[p][rbase + j, pl.ds(0, 16)]
                    v2 = rbuf[p][rbase + j, pl.ds(16, 16)]
                    plsc.store_scatter(obuf[p], [obase + j + cvec], v1)
                    plsc.store_scatter(obuf[p], [obase + j + c2vec], v2,
                                       mask=m10)
                return c
            lax.fori_loop(0, _GB, bl_body, 0)

            def mv_body(v, c):
                s = pl.ds(v * 16, 16)
                rows = lanes + v * 16
                mbuf[p][s] = plsc.load_gather(rbuf[p], [rows, col_mean])
                vbuf[p][s] = plsc.load_gather(rbuf[p], [rows, col_lv])
                return c
            lax.fori_loop(0, _GROUP // 16, mv_body, 0)

        def out_slices(g):
            base_b = b0 + g * _GB
            return (dw_hbm.at[pl.ds(base_b * _D * _D, _OUTW)],
                    mean_hbm.at[pl.ds(base_b * _D, _GROUP)],
                    lv_hbm.at[pl.ds(base_b * _D, _GROUP)])

        def fire_out(g, p):
            dws, ms, vs = out_slices(g)
            pltpu.async_copy(obuf[p], dws, sout[p])
            pltpu.async_copy(mbuf[p], ms, sout[p])
            pltpu.async_copy(vbuf[p], vs, sout[p])

        def wait_out(g, p):
            dws, ms, vs = out_slices(g)
            pltpu.make_async_copy(obuf[p], dws, sout[p]).wait()
            pltpu.make_async_copy(mbuf[p], ms, sout[p]).wait()
            pltpu.make_async_copy(vbuf[p], vs, sout[p]).wait()

        fire_labels(0, 0)
        fire_labels(1, 1)
        wait_labels(0, 0)
        compute_idx(0)
        fire_gathers(0)

        def halfstep(g, p, q):
            @pl.when(g + 1 <= _NG - 1)
            def _():
                wait_labels(g + 1, q)
                compute_idx(q)
                fire_gathers(q)
            wait_gathers(p)

            @pl.when(g >= 2)
            def _():
                wait_out(g - 2, p)
            transpose(p)
            fire_out(g, p)

            @pl.when(g + 2 <= _NG - 1)
            def _():
                fire_labels(g + 2, p)

        def step(gg, c):
            halfstep(2 * gg, 0, 1)
            halfstep(2 * gg + 1, 1, 0)
            return c
        lax.fori_loop(0, _NG // 2, step, 0)
        wait_out(_NG - 2, (_NG - 2) % 2)
        wait_out(_NG - 1, (_NG - 1) % 2)

    return k(table, labels_flat, jpat)


def kernel(labels, mean, log_var, domain_weights):
    labels = labels.astype(jnp.int32)
    tdw = jnp.transpose(domain_weights, (1, 2, 0))
    table = jnp.concatenate(
        [tdw, mean[:, :, None], log_var[:, :, None],
         jnp.zeros((_D, _K, _RW - _D - 2), jnp.float32)], axis=2)
    table = table.reshape(_D * _K, _RW)
    dwf, mf, vf = _sc_gather(table, labels.reshape(-1), jnp.asarray(_JPAT))
    return (mf.reshape(_B, _D), vf.reshape(_B, _D),
            dwf.reshape(_B, _D, _D))
